# Initial kernel scaffold; baseline (speedup 1.0000x reference)
import jax
import jax.numpy as jnp
from jax.experimental import pallas as pl


def kernel(x, embed_param):
    raise NotImplementedError("write your pallas kernel here")


# speedup vs baseline: 5.0678x; 5.0678x over previous
import functools

import jax
import jax.numpy as jnp
from jax import lax
from jax.experimental import pallas as pl
from jax.experimental.pallas import tpu as pltpu
from jax.experimental.pallas import tpu_sc as plsc

B, T = 16384, 200
V = 260
N = B * T
D = 64
NC, NS = 2, 16
NW = NC * NS
ROWS_PER_W = N // NW
BLK = 512
IDX_ROWS = BLK // 128
NBLK = ROWS_PER_W // BLK

_mesh = plsc.VectorSubcoreMesh(core_axis_name="c", subcore_axis_name="s")


@functools.partial(
    pl.kernel,
    out_type=jax.ShapeDtypeStruct((N, D), jnp.float32),
    mesh=_mesh,
    compiler_params=pltpu.CompilerParams(use_tc_tiling_on_sc=False),
    scratch_types=[
        pltpu.VMEM((IDX_ROWS, 128), jnp.int32),
        pltpu.VMEM((BLK, D), jnp.float32),
        pltpu.VMEM((V, D), jnp.float32),
        pltpu.VMEM_SHARED((V, D), jnp.float32),
        pltpu.SemaphoreType.DMA,
    ],
)
def _embed_sc(x_hbm, table_hbm, out_hbm, idx_v, rows_v, table_v, table_sh, sem):
    wid = lax.axis_index("s") * NC + lax.axis_index("c")
    row_base = wid * ROWS_PER_W

    pltpu.sync_copy(table_hbm, table_v)
    pltpu.sync_copy(table_v, table_sh)
    plsc.subcore_barrier()

    def body(i, carry):
        off = pl.multiple_of(row_base + i * BLK, BLK)
        pltpu.sync_copy(x_hbm.at[pl.ds(pl.multiple_of(off // 128, IDX_ROWS), IDX_ROWS)], idx_v)
        copies = [
            pltpu.async_copy(
                table_sh.at[idx_v.at[j]],
                rows_v.at[pl.ds(j * 128, 128)],
                sem,
            )
            for j in range(IDX_ROWS)
        ]
        for c in copies:
            c.wait()
        pltpu.sync_copy(rows_v, out_hbm.at[pl.ds(off, BLK)])
        return carry

    lax.fori_loop(0, NBLK, body, 0)


def kernel(x, embed_param):
    xf = x.reshape(N // 128, 128)
    out = _embed_sc(xf, embed_param)
    return out.reshape(B, T, D)


# speedup vs baseline: 5.9798x; 1.1800x over previous
import functools

import jax
import jax.numpy as jnp
from jax import lax
from jax.experimental import pallas as pl
from jax.experimental.pallas import tpu as pltpu
from jax.experimental.pallas import tpu_sc as plsc

B, T = 16384, 200
V = 260
D = 64
NC, NS = 2, 16
NW = NC * NS
NU = T * (D // 8)
UPW = NU // NW
NBC = B // 128
CHUNK = 16
NCHUNK = NBC // CHUNK

_mesh = plsc.VectorSubcoreMesh(core_axis_name="c", subcore_axis_name="s")


@functools.partial(
    pl.kernel,
    out_type=jax.ShapeDtypeStruct((NU, NBC, 8, 128), jnp.float32),
    mesh=_mesh,
    compiler_params=pltpu.CompilerParams(
        use_tc_tiling_on_sc=False, needs_layout_passes=False
    ),
    scratch_types=[
        pltpu.VMEM((B,), jnp.int32),
        pltpu.VMEM((2, CHUNK, 8, 128), jnp.float32),
        pltpu.VMEM((D, V), jnp.float32),
        pltpu.SemaphoreType.DMA,
        pltpu.SemaphoreType.DMA,
    ],
)
def _embed_sc(xt_hbm, tab_hbm, out_hbm, idx_v, stage_v, tab_v, sem0, sem1):
    wid = lax.axis_index("s") * NC + lax.axis_index("c")
    u0 = wid * UPW

    pltpu.sync_copy(tab_hbm, tab_v)
    sems = (sem0, sem1)

    def unit_body(k, carry):
        u = u0 + k
        t = u // 8
        dr8 = (u % 8) * 8
        rowv = [jnp.full((16,), dr8 + d_, jnp.int32) for d_ in range(8)]

        pltpu.sync_copy(xt_hbm.at[t], idx_v)

        copies = [None, None]
        for chunk in range(NCHUNK):
            buf = chunk % 2
            if copies[buf] is not None:
                copies[buf].wait()

            def bc_body(bi, c2, chunk=chunk, buf=buf):
                bc = chunk * CHUNK + bi
                base = bc * 128
                for b16 in range(8):
                    idx16 = idx_v[pl.ds(base + b16 * 16, 16)]
                    for d_ in range(8):
                        vals = plsc.load_gather(tab_v, [rowv[d_], idx16])
                        stage_v[buf, bi, d_, pl.ds(b16 * 16, 16)] = vals
                return c2

            lax.fori_loop(0, CHUNK, bc_body, 0)
            copies[buf] = pltpu.async_copy(
                stage_v.at[buf],
                out_hbm.at[u, pl.ds(chunk * CHUNK, CHUNK)],
                sems[buf],
            )
        for c in copies:
            c.wait()
        return carry

    lax.fori_loop(0, UPW, unit_body, 0)


def kernel(x, embed_param):
    xt = x.T.reshape(T, B)
    tab = embed_param.T.reshape(D, V)
    out5 = _embed_sc(xt, tab)
    out = out5.reshape(T, 8, NBC, 8, 128)
    out = out.transpose(2, 4, 0, 1, 3)
    return out.reshape(B, T, D)
